# baseline (device time: 36464 ns/iter reference)
import jax
import jax.numpy as jnp
from jax import lax
from jax.experimental import pallas as pl
from jax.experimental.pallas import tpu as pltpu


def kernel(x, assign, W1, W2):
    t, d = x.shape
    assign2 = assign.reshape(t, 1)

    def body(x_ref, a_ref, w1_ref, w2_ref, out_ref,
             xr_ref, accs_ref, accr_ref, send_sems, recv_sems):
        my_x = lax.axis_index("x")
        my_y = lax.axis_index("y")
        my_z = lax.axis_index("z")
        peer = (1 - my_x, my_y, my_z)

        barrier = pltpu.get_barrier_semaphore()
        pl.semaphore_signal(barrier, inc=1, device_id=peer,
                            device_id_type=pl.DeviceIdType.MESH)
        pl.semaphore_wait(barrier, 1)

        rdma_x = pltpu.make_async_remote_copy(
            src_ref=x_ref, dst_ref=xr_ref,
            send_sem=send_sems.at[0], recv_sem=recv_sems.at[0],
            device_id=peer, device_id_type=pl.DeviceIdType.MESH)
        rdma_x.start()
        rdma_x.wait_recv()

        accs_ref[...] = xr_ref[...] * 2.0

        rdma_acc = pltpu.make_async_remote_copy(
            src_ref=accs_ref, dst_ref=accr_ref,
            send_sem=send_sems.at[1], recv_sem=recv_sems.at[1],
            device_id=peer, device_id_type=pl.DeviceIdType.MESH)
        rdma_acc.start()
        rdma_acc.wait_recv()

        out_ref[...] = accr_ref[...]

        rdma_x.wait_send()
        rdma_acc.wait_send()

    return pl.pallas_call(
        body,
        out_shape=jax.ShapeDtypeStruct((t, d), jnp.float32),
        in_specs=[pl.BlockSpec(memory_space=pltpu.VMEM)] * 4,
        out_specs=pl.BlockSpec(memory_space=pltpu.VMEM),
        scratch_shapes=[
            pltpu.VMEM((t, d), jnp.float32),
            pltpu.VMEM((t, d), jnp.float32),
            pltpu.VMEM((t, d), jnp.float32),
            pltpu.SemaphoreType.DMA((2,)),
            pltpu.SemaphoreType.DMA((2,)),
        ],
        compiler_params=pltpu.CompilerParams(collective_id=0),
    )(x, assign2, W1, W2)


# device time: 25751 ns/iter; 1.4160x vs baseline; 1.4160x over previous
import jax
import jax.numpy as jnp
from jax import lax
from jax.experimental import pallas as pl
from jax.experimental.pallas import tpu as pltpu

N_EXP_LOCAL = 2
N_CHUNK = 4


def kernel(x, assign, W1, W2):
    t, d = x.shape
    assign2 = assign.reshape(t, 1)
    rows = t // N_CHUNK
    S_A, S_X, S_R = 0, 1, 1 + N_CHUNK

    def body(x_ref, a_ref, w1_ref, w2_ref, out_ref,
             xs_ref, xr_ref, ar_ref, accs_ref, accr_ref,
             send_sems, recv_sems):
        my_x = lax.axis_index("x")
        my_y = lax.axis_index("y")
        my_z = lax.axis_index("z")
        peer = (1 - my_x, my_y, my_z)

        barrier = pltpu.get_barrier_semaphore()
        pl.semaphore_signal(barrier, inc=1, device_id=peer,
                            device_id_type=pl.DeviceIdType.MESH)
        pl.semaphore_wait(barrier, 1)

        xs_ref[...] = x_ref[...].astype(jnp.bfloat16)
        rdma_a = pltpu.make_async_remote_copy(
            src_ref=a_ref, dst_ref=ar_ref,
            send_sem=send_sems.at[S_A], recv_sem=recv_sems.at[S_A],
            device_id=peer, device_id_type=pl.DeviceIdType.MESH)
        rdma_a.start()
        rdma_xs = []
        for c in range(N_CHUNK):
            sl = pl.ds(c * rows, rows)
            r = pltpu.make_async_remote_copy(
                src_ref=xs_ref.at[sl, :], dst_ref=xr_ref.at[sl, :],
                send_sem=send_sems.at[S_X + c], recv_sem=recv_sems.at[S_X + c],
                device_id=peer, device_id_type=pl.DeviceIdType.MESH)
            r.start()
            rdma_xs.append(r)

        def expert_contrib(tok, asn, e_loc):
            e_glob = my_x * N_EXP_LOCAL + e_loc
            xe = jnp.where(asn == e_glob, tok, 0.0)
            h = jnp.maximum(
                jnp.dot(xe, w1_ref[e_loc], preferred_element_type=jnp.float32),
                0.0)
            return jnp.dot(h, w2_ref[e_loc], preferred_element_type=jnp.float32)

        mine = expert_contrib(x_ref[...], a_ref[...], 0)
        mine = mine + expert_contrib(x_ref[...], a_ref[...], 1)
        out_ref[...] = mine

        rdma_a.wait_recv()

        rdma_rets = []
        for c in range(N_CHUNK):
            sl = pl.ds(c * rows, rows)
            rdma_xs[c].wait_recv()
            tok = xr_ref[sl, :].astype(jnp.float32)
            acc = expert_contrib(tok, ar_ref[sl, :], 0)
            acc = acc + expert_contrib(tok, ar_ref[sl, :], 1)
            accs_ref[sl, :] = acc.astype(jnp.bfloat16)
            r = pltpu.make_async_remote_copy(
                src_ref=accs_ref.at[sl, :], dst_ref=accr_ref.at[sl, :],
                send_sem=send_sems.at[S_R + c], recv_sem=recv_sems.at[S_R + c],
                device_id=peer, device_id_type=pl.DeviceIdType.MESH)
            r.start()
            rdma_rets.append(r)

        for c in range(N_CHUNK):
            sl = pl.ds(c * rows, rows)
            rdma_rets[c].wait_recv()
            out_ref[sl, :] = out_ref[sl, :] + accr_ref[sl, :].astype(jnp.float32)

        rdma_a.wait_send()
        for r in rdma_xs:
            r.wait_send()
        for r in rdma_rets:
            r.wait_send()

    n_sems = 1 + 2 * N_CHUNK
    return pl.pallas_call(
        body,
        out_shape=jax.ShapeDtypeStruct((t, d), jnp.float32),
        in_specs=[pl.BlockSpec(memory_space=pltpu.VMEM)] * 4,
        out_specs=pl.BlockSpec(memory_space=pltpu.VMEM),
        scratch_shapes=[
            pltpu.VMEM((t, d), jnp.bfloat16),
            pltpu.VMEM((t, d), jnp.bfloat16),
            pltpu.VMEM((t, 1), jnp.int32),
            pltpu.VMEM((t, d), jnp.bfloat16),
            pltpu.VMEM((t, d), jnp.bfloat16),
            pltpu.SemaphoreType.DMA((n_sems,)),
            pltpu.SemaphoreType.DMA((n_sems,)),
        ],
        compiler_params=pltpu.CompilerParams(collective_id=0),
    )(x, assign2, W1, W2)
